# Initial kernel scaffold; baseline (speedup 1.0000x reference)
import jax
import jax.numpy as jnp
from jax.experimental import pallas as pl


def kernel(token_ids, weight):
    raise NotImplementedError("write your pallas kernel here")


# speedup vs baseline: 1.0948x; 1.0948x over previous
import functools

import jax
import jax.numpy as jnp
from jax import lax
from jax.experimental import pallas as pl
from jax.experimental.pallas import tpu as pltpu
from jax.experimental.pallas import tpu_sc as plsc

_NC = 2
_NS = 16
_NW = _NC * _NS


def kernel(token_ids, weight):
    B0, S = token_ids.shape
    V, D = weight.shape
    B = B0 * S
    idx = token_ids.reshape(B)
    b_per_w = B // _NW
    CH = 1024
    n_ch = b_per_w // CH

    mesh = plsc.VectorSubcoreMesh(core_axis_name="c", subcore_axis_name="s")

    @functools.partial(
        pl.kernel,
        mesh=mesh,
        out_type=jax.ShapeDtypeStruct((B, D), jnp.float32),
        compiler_params=pltpu.CompilerParams(use_tc_tiling_on_sc=False),
        scratch_types=[
            pltpu.VMEM((CH,), jnp.int32),
            pltpu.VMEM((CH, D), jnp.float32),
            pltpu.SemaphoreType.DMA,
        ],
    )
    def gather_kernel(idx_hbm, table_hbm, out_hbm, idx_v, rows_v, sem):
        wid = lax.axis_index("s") * _NC + lax.axis_index("c")
        base = wid * b_per_w

        @pl.loop(0, n_ch)
        def _(i):
            off = base + i * CH
            pltpu.sync_copy(idx_hbm.at[pl.ds(off, CH)], idx_v)
            pltpu.async_copy(table_hbm.at[idx_v], rows_v, sem).wait()
            pltpu.sync_copy(rows_v, out_hbm.at[pl.ds(off, CH)])

    out = gather_kernel(idx, weight)
    return out.reshape(B0, S, D)


# speedup vs baseline: 1.1099x; 1.0138x over previous
import functools

import jax
import jax.numpy as jnp
from jax import lax
from jax.experimental import pallas as pl
from jax.experimental.pallas import tpu as pltpu
from jax.experimental.pallas import tpu_sc as plsc

_NC = 2
_NS = 16
_NW = _NC * _NS


def kernel(token_ids, weight):
    B0, S = token_ids.shape
    V, D = weight.shape
    B = B0 * S
    idx = token_ids.reshape(B)
    b_per_w = B // _NW
    CH = 1600
    n_ch = b_per_w // CH

    mesh = plsc.VectorSubcoreMesh(core_axis_name="c", subcore_axis_name="s")

    @functools.partial(
        pl.kernel,
        mesh=mesh,
        out_type=jax.ShapeDtypeStruct((B, D), jnp.float32),
        compiler_params=pltpu.CompilerParams(use_tc_tiling_on_sc=False),
        scratch_types=[
            pltpu.VMEM((b_per_w,), jnp.int32),
            pltpu.VMEM((2, CH, D), jnp.float32),
            pltpu.SemaphoreType.DMA((2,)),
            pltpu.SemaphoreType.DMA((2,)),
        ],
    )
    def gather_kernel(idx_hbm, table_hbm, out_hbm, idx_v, rows_v, gsem, wsem):
        wid = lax.axis_index("s") * _NC + lax.axis_index("c")
        base = wid * b_per_w

        def g_copy(i, h):
            return pltpu.make_async_copy(
                table_hbm.at[idx_v.at[pl.ds(i * CH, CH)]],
                rows_v.at[h],
                gsem.at[h],
            )

        def w_copy(i, h):
            return pltpu.make_async_copy(
                rows_v.at[h],
                out_hbm.at[pl.ds(base + i * CH, CH)],
                wsem.at[h],
            )

        pltpu.sync_copy(idx_hbm.at[pl.ds(base, b_per_w)], idx_v)

        g_copy(0, 0).start()
        g_copy(0, 0).wait()
        w_copy(0, 0).start()
        g_copy(1, 1).start()

        def body(i, h):
            g_copy(i, h).wait()
            w_copy(i, h).start()
            w_copy(i - 1, 1 - h).wait()
            g_copy(i + 1, 1 - h).start()

        @pl.loop(1, n_ch - 1, step=2)
        def _(g):
            body(g, 1)
            body(g + 1, 0)

        g_copy(n_ch - 1, 1).wait()
        w_copy(n_ch - 1, 1).start()
        w_copy(n_ch - 2, 0).wait()
        w_copy(n_ch - 1, 1).wait()

    out = gather_kernel(idx, weight)
    return out.reshape(B0, S, D)


# speedup vs baseline: 1.6338x; 1.4720x over previous
import functools

import jax
import jax.numpy as jnp
from jax import lax
from jax.experimental import pallas as pl
from jax.experimental.pallas import tpu as pltpu
from jax.experimental.pallas import tpu_sc as plsc

_NW = 32


def kernel(token_ids, weight):
    B0, S = token_ids.shape
    V, D = weight.shape
    B = B0 * S
    idx = token_ids.reshape(B)
    BPT = B0 // _NW
    IPT = BPT * S
    NBT = BPT // 128
    NDT = D // 8
    SEC = 128 * 8 * NBT

    mesh = plsc.VectorSubcoreMesh(core_axis_name="c", subcore_axis_name="s")

    @functools.partial(
        pl.kernel,
        mesh=mesh,
        out_type=jax.ShapeDtypeStruct((S, B0 * D), jnp.float32),
        compiler_params=pltpu.CompilerParams(
            use_tc_tiling_on_sc=False, needs_layout_passes=False),
        scratch_types=[
            pltpu.VMEM((IPT,), jnp.int32),
            pltpu.VMEM((2, BPT), jnp.int32),
            pltpu.VMEM((2, BPT, D), jnp.float32),
            pltpu.VMEM((2, NDT * SEC), jnp.float32),
            pltpu.SemaphoreType.DMA((2,)),
            pltpu.SemaphoreType.DMA((2,)),
        ],
    )
    def gather_kernel(idx_hbm, table_hbm, z_hbm, idx_all, idx_s, chunk, zbuf,
                      gsem, wsem):
        wid = lax.axis_index("s") * 2 + lax.axis_index("c")
        ibase = wid * IPT
        bt0 = wid * NBT

        iota = lax.iota(jnp.int32, 16)

        def build_idx(s, h):
            @pl.loop(0, BPT // 16)
            def _(k):
                pos = (k * 16 + iota) * S + s
                vals = plsc.load_gather(idx_all, [pos])
                idx_s.at[h][pl.ds(k * 16, 16)] = vals

        def g_copy(h):
            return pltpu.make_async_copy(
                table_hbm.at[idx_s.at[h]], chunk.at[h], gsem.at[h])

        def w_copy(s, h, dt):
            return pltpu.make_async_copy(
                zbuf.at[h].at[pl.ds(dt * SEC, SEC)],
                z_hbm.at[s].at[pl.ds((dt * 128 + bt0) * 1024, SEC)],
                wsem.at[h],
            )

        def out_start(s, h):
            for dt in range(NDT):
                w_copy(s, h, dt).start()

        def out_wait(s, h):
            for dt in range(NDT):
                w_copy(s, h, dt).wait()

        def transpose(h):
            @pl.loop(0, NBT * 8)
            def _(k):
                btl = k >> 3
                blg = k & 7
                rows = btl * 128 + blg * 16 + iota
                off_b = btl * 1024 + blg * 16
                for dt in range(NDT):
                    for dr in range(8):
                        col = jnp.full((16,), dt * 8 + dr, dtype=jnp.int32)
                        v = plsc.load_gather(chunk.at[h], [rows, col])
                        zbuf.at[h][pl.ds(off_b + dt * SEC + dr * 128, 16)] = v

        pltpu.sync_copy(idx_hbm.at[pl.ds(ibase, IPT)], idx_all)

        def step(s, h, prefetch, outwait):
            if prefetch:
                build_idx(s + 1, 1 - h)
                g_copy(1 - h).start()
            g_copy(h).wait()
            if outwait:
                out_wait(s - 2, h)
            transpose(h)
            out_start(s, h)

        build_idx(0, 0)
        g_copy(0).start()
        step(0, 0, True, False)
        step(1, 1, True, False)

        @pl.loop(2, S - 2, step=2)
        def _(g):
            step(g, 0, True, True)
            step(g + 1, 1, True, True)

        step(S - 2, 0, True, True)
        step(S - 1, 1, False, True)
        out_wait(S - 2, 0)
        out_wait(S - 1, 1)

    z = gather_kernel(idx, weight)
    out = (z.reshape(S, NDT, 128, 8, 128)
            .transpose(2, 4, 0, 1, 3)
            .reshape(B0, S, D))
    return out


# speedup vs baseline: 2.0033x; 1.2261x over previous
import functools

import jax
import jax.numpy as jnp
from jax import lax
from jax.experimental import pallas as pl
from jax.experimental.pallas import tpu as pltpu
from jax.experimental.pallas import tpu_sc as plsc

_NW = 32


def kernel(token_ids, weight):
    B0, S = token_ids.shape
    V, D = weight.shape
    B = B0 * S
    idx = token_ids.reshape(B)
    BPT = B0 // _NW
    IPT = BPT * S
    NBT = BPT // 128
    NDT = D // 8
    SEC = 128 * 8 * NBT

    mesh = plsc.VectorSubcoreMesh(core_axis_name="c", subcore_axis_name="s")

    @functools.partial(
        pl.kernel,
        mesh=mesh,
        out_type=jax.ShapeDtypeStruct((S, B0 * D), jnp.float32),
        compiler_params=pltpu.CompilerParams(
            use_tc_tiling_on_sc=False, needs_layout_passes=False),
        scratch_types=[
            pltpu.VMEM((IPT,), jnp.int32),
            pltpu.VMEM((2, BPT), jnp.int32),
            pltpu.VMEM((2, BPT, D), jnp.float32),
            pltpu.VMEM((2, NDT * SEC), jnp.float32),
            pltpu.SemaphoreType.DMA((2,)),
            pltpu.SemaphoreType.DMA((2,)),
        ],
    )
    def gather_kernel(idx_hbm, table_hbm, z_hbm, idx_all, idx_s, chunk, zbuf,
                      gsem, wsem):
        wid = lax.axis_index("s") * 2 + lax.axis_index("c")
        ibase = wid * IPT
        bt0 = wid * NBT

        iota = lax.iota(jnp.int32, 16)

        def build_idx(s, h):
            @plsc.parallel_loop(0, BPT // 16, unroll=4)
            def _(k):
                pos = (k * 16 + iota) * S + s
                vals = plsc.load_gather(idx_all, [pos])
                idx_s.at[h][pl.ds(k * 16, 16)] = vals

        def g_copy(h):
            return pltpu.make_async_copy(
                table_hbm.at[idx_s.at[h]], chunk.at[h], gsem.at[h])

        def w_copy(s, h, dt):
            return pltpu.make_async_copy(
                zbuf.at[h].at[pl.ds(dt * SEC, SEC)],
                z_hbm.at[s].at[pl.ds((dt * 128 + bt0) * 1024, SEC)],
                wsem.at[h],
            )

        def out_start(s, h):
            for dt in range(NDT):
                w_copy(s, h, dt).start()

        def out_wait(s, h):
            for dt in range(NDT):
                w_copy(s, h, dt).wait()

        def transpose(h):
            @plsc.parallel_loop(0, NBT * 8, unroll=4)
            def _(k):
                btl = k >> 3
                blg = k & 7
                rows = btl * 128 + blg * 16 + iota
                off_b = btl * 1024 + blg * 16
                for dt in range(NDT):
                    for dr in range(8):
                        col = jnp.full((16,), dt * 8 + dr, dtype=jnp.int32)
                        v = plsc.load_gather(chunk.at[h], [rows, col])
                        zbuf.at[h][pl.ds(off_b + dt * SEC + dr * 128, 16)] = v

        pltpu.sync_copy(idx_hbm.at[pl.ds(ibase, IPT)], idx_all)

        def step(s, h, prefetch, outwait):
            if prefetch:
                build_idx(s + 1, 1 - h)
                g_copy(1 - h).start()
            g_copy(h).wait()
            if outwait:
                out_wait(s - 2, h)
            transpose(h)
            out_start(s, h)

        build_idx(0, 0)
        g_copy(0).start()
        step(0, 0, True, False)
        step(1, 1, True, False)

        @pl.loop(2, S - 2, step=2)
        def _(g):
            step(g, 0, True, True)
            step(g + 1, 1, True, True)

        step(S - 2, 0, True, True)
        step(S - 1, 1, False, True)
        out_wait(S - 2, 0)
        out_wait(S - 1, 1)

    z = gather_kernel(idx, weight)
    out = (z.reshape(S, NDT, 128, 8, 128)
            .transpose(2, 4, 0, 1, 3)
            .reshape(B0, S, D))
    return out
